# baseline (speedup 1.0000x reference)
import functools

import jax
import jax.numpy as jnp
from jax import lax
from jax.experimental import pallas as pl
from jax.experimental.pallas import tpu as pltpu
from jax.experimental.pallas import tpu_sc as plsc

N_NODES = 10000
N_EDGES = 160000
D = 256
DW = D // 2
HID = 512

NC = 2
NS = 16
NW = NC * NS
NSPLIT = 1
E_SPLIT = N_EDGES // NSPLIT
E_PER_W = E_SPLIT // NW
CHUNK = 40
N_CHUNKS = E_PER_W // CHUNK
LANES = 16


def _make_relay():
    mesh = plsc.VectorSubcoreMesh(core_axis_name="c", subcore_axis_name="s")

    @functools.partial(
        pl.kernel,
        mesh=mesh,
        out_type=(
            jax.ShapeDtypeStruct((E_SPLIT, DW), jnp.int32),
            jax.ShapeDtypeStruct((E_SPLIT, DW), jnp.int32),
        ),
        compiler_params=pltpu.CompilerParams(
            use_tc_tiling_on_sc=False,
        ),
        scratch_types=[
            pltpu.VMEM((N_CHUNKS, CHUNK), jnp.int32),
            pltpu.VMEM((N_CHUNKS, CHUNK), jnp.int32),
            pltpu.VMEM((2, CHUNK, DW), jnp.int32),
            pltpu.VMEM((2, CHUNK, DW), jnp.int32),
            pltpu.SemaphoreType.DMA,
            pltpu.SemaphoreType.DMA,
            pltpu.SemaphoreType.DMA,
            pltpu.SemaphoreType.DMA,
            pltpu.SemaphoreType.DMA,
            pltpu.SemaphoreType.DMA,
            pltpu.SemaphoreType.DMA,
            pltpu.SemaphoreType.DMA,
        ],
    )
    def relay(z32_hbm, idx0_hbm, idx1_hbm, src_out, dst_out,
              idx0_v, idx1_v, a_v, b_v,
              sga0, sga1, sgb0, sgb1, swa0, swa1, swb0, swb1):
        wid = lax.axis_index("s") * NC + lax.axis_index("c")
        base = wid * E_PER_W
        sga = (sga0, sga1)
        sgb = (sgb0, sgb1)
        swa = (swa0, swa1)
        swb = (swb0, swb1)

        pltpu.sync_copy(idx0_hbm.at[wid], idx0_v)
        pltpu.sync_copy(idx1_hbm.at[wid], idx1_v)

        def g_start(ci, p):
            pltpu.async_copy(z32_hbm.at[idx0_v.at[ci]], a_v.at[p], sga[p])
            pltpu.async_copy(z32_hbm.at[idx1_v.at[ci]], b_v.at[p], sgb[p])

        def g_wait(ci, p):
            pltpu.make_async_copy(
                z32_hbm.at[idx0_v.at[ci]], a_v.at[p], sga[p]).wait()
            pltpu.make_async_copy(
                z32_hbm.at[idx1_v.at[ci]], b_v.at[p], sgb[p]).wait()

        def osl(out, ci):
            return out.at[pl.ds(base + ci * CHUNK, CHUNK)]

        def wb_start(ci, p):
            pltpu.async_copy(a_v.at[p], osl(src_out, ci), swa[p])
            pltpu.async_copy(b_v.at[p], osl(dst_out, ci), swb[p])

        def wb_wait(ci, p):
            pltpu.make_async_copy(a_v.at[p], osl(src_out, ci), swa[p]).wait()
            pltpu.make_async_copy(b_v.at[p], osl(dst_out, ci), swb[p]).wait()

        n_pairs = (N_CHUNKS - 1) // 2
        g_start(0, 0)
        g_start(1, 1)

        def pair_body(i, carry):
            g0 = 2 * i
            g_wait(g0, 0)
            wb_start(g0, 0)
            wb_wait(g0, 0)
            g_start(g0 + 2, 0)

            g_wait(g0 + 1, 1)
            wb_start(g0 + 1, 1)
            wb_wait(g0 + 1, 1)

            @pl.when(i < n_pairs - 1)
            def _():
                g_start(g0 + 3, 1)
            return carry

        lax.fori_loop(0, n_pairs, pair_body, 0)

        last = N_CHUNKS - 1
        g_wait(last, 0)
        wb_start(last, 0)
        wb_wait(last, 0)

    return relay


_relay_cache = []


def _relay(z32, idx0, idx1):
    if not _relay_cache:
        _relay_cache.append(_make_relay())
    return _relay_cache[0](z32, idx0, idx1)


BE = 8000


def _mlp_body(s_ref, d_ref, w1_ref, b1_ref, w2t_ref, b2_ref, o_ref):
    ws = s_ref[...]
    wd = d_ref[...]
    himask = jnp.int32(-65536)
    s_lo = lax.bitcast_convert_type(lax.shift_left(ws, 16), jnp.float32)
    s_hi = lax.bitcast_convert_type(ws & himask, jnp.float32)
    d_lo = lax.bitcast_convert_type(lax.shift_left(wd, 16), jnp.float32)
    d_hi = lax.bitcast_convert_type(wd & himask, jnp.float32)
    xe = (s_lo * d_lo).astype(jnp.bfloat16)
    xo = (s_hi * d_hi).astype(jnp.bfloat16)
    w1 = w1_ref[...]
    h = jnp.dot(xe, w1[:DW], preferred_element_type=jnp.float32)
    h += jnp.dot(xo, w1[DW:], preferred_element_type=jnp.float32)
    h = jnp.maximum(h + b1_ref[...], 0.0)
    logits = jnp.sum(h * w2t_ref[...], axis=1, keepdims=True) + b2_ref[...]
    o_ref[...] = jax.nn.sigmoid(logits)


def _mlp(s32, d32, W1eo, b1, W2, b2):
    grid = E_SPLIT // BE
    return pl.pallas_call(
        _mlp_body,
        grid=(grid,),
        in_specs=[
            pl.BlockSpec((BE, DW), lambda i: (i, 0)),
            pl.BlockSpec((BE, DW), lambda i: (i, 0)),
            pl.BlockSpec((D, HID), lambda i: (0, 0)),
            pl.BlockSpec((1, HID), lambda i: (0, 0)),
            pl.BlockSpec((1, HID), lambda i: (0, 0)),
            pl.BlockSpec((1, 1), lambda i: (0, 0)),
        ],
        out_specs=pl.BlockSpec((BE, 1), lambda i: (i, 0)),
        out_shape=jax.ShapeDtypeStruct((E_SPLIT, 1), jnp.float32),
        compiler_params=pltpu.CompilerParams(
            dimension_semantics=("arbitrary",),
        ),
    )(s32, d32, W1eo, b1, W2, b2)


def kernel(z, edge, W1, b1, W2, b2):
    zb = z.astype(jnp.bfloat16)
    z32 = lax.bitcast_convert_type(zb.reshape(N_NODES, DW, 2), jnp.int32)
    edge = edge.astype(jnp.int32)
    idx0 = edge[0].reshape(NSPLIT, NW, N_CHUNKS, CHUNK)
    idx1 = edge[1].reshape(NSPLIT, NW, N_CHUNKS, CHUNK)
    w1eo = jnp.concatenate([W1[0::2], W1[1::2]], axis=0).astype(jnp.bfloat16)
    b1r = b1.reshape(1, HID)
    w2r = W2.reshape(1, HID)
    b2r = b2.reshape(1, 1)
    outs = []
    for k in range(NSPLIT):
        s32, d32 = _relay(z32, idx0[k], idx1[k])
        outs.append(_mlp(s32, d32, w1eo, b1r, w2r, b2r))
    if NSPLIT == 1:
        return outs[0]
    return jnp.concatenate(outs, axis=0)


# speedup vs baseline: 1.0511x; 1.0511x over previous
import functools

import jax
import jax.numpy as jnp
from jax import lax
from jax.experimental import pallas as pl
from jax.experimental.pallas import tpu as pltpu
from jax.experimental.pallas import tpu_sc as plsc

N_NODES = 10000
N_EDGES = 160000
D = 256
DW = D // 2
HID = 512

NC = 2
NS = 16
NW = NC * NS
NSPLIT = 1
E_SPLIT = N_EDGES // NSPLIT
E_PER_W = E_SPLIT // NW
CHUNK = 40
N_CHUNKS = E_PER_W // CHUNK
LANES = 16


def _make_relay():
    mesh = plsc.VectorSubcoreMesh(core_axis_name="c", subcore_axis_name="s")

    @functools.partial(
        pl.kernel,
        mesh=mesh,
        out_type=(
            jax.ShapeDtypeStruct((E_SPLIT, DW), jnp.int32),
            jax.ShapeDtypeStruct((E_SPLIT, DW), jnp.int32),
        ),
        compiler_params=pltpu.CompilerParams(
            use_tc_tiling_on_sc=False,
        ),
        scratch_types=[
            pltpu.VMEM((N_CHUNKS, CHUNK), jnp.int32),
            pltpu.VMEM((N_CHUNKS, CHUNK), jnp.int32),
            pltpu.VMEM((2, CHUNK, DW), jnp.int32),
            pltpu.VMEM((2, CHUNK, DW), jnp.int32),
            pltpu.SemaphoreType.DMA,
            pltpu.SemaphoreType.DMA,
            pltpu.SemaphoreType.DMA,
            pltpu.SemaphoreType.DMA,
            pltpu.SemaphoreType.DMA,
            pltpu.SemaphoreType.DMA,
            pltpu.SemaphoreType.DMA,
            pltpu.SemaphoreType.DMA,
        ],
    )
    def relay(z32_hbm, idx0_hbm, idx1_hbm, src_out, dst_out,
              idx0_v, idx1_v, a_v, b_v,
              sga0, sga1, sgb0, sgb1, swa0, swa1, swb0, swb1):
        wid = lax.axis_index("s") * NC + lax.axis_index("c")
        base = wid * E_PER_W
        sga = (sga0, sga1)
        sgb = (sgb0, sgb1)
        swa = (swa0, swa1)
        swb = (swb0, swb1)

        pltpu.sync_copy(idx0_hbm.at[wid], idx0_v)
        pltpu.sync_copy(idx1_hbm.at[wid], idx1_v)

        def g_start(ci, p):
            pltpu.async_copy(z32_hbm.at[idx0_v.at[ci]], a_v.at[p], sga[p])
            pltpu.async_copy(z32_hbm.at[idx1_v.at[ci]], b_v.at[p], sgb[p])

        def g_wait(ci, p):
            pltpu.make_async_copy(
                z32_hbm.at[idx0_v.at[ci]], a_v.at[p], sga[p]).wait()
            pltpu.make_async_copy(
                z32_hbm.at[idx1_v.at[ci]], b_v.at[p], sgb[p]).wait()

        def osl(out, ci):
            return out.at[pl.ds(base + ci * CHUNK, CHUNK)]

        def wb_start(ci, p):
            pltpu.async_copy(a_v.at[p], osl(src_out, ci), swa[p])
            pltpu.async_copy(b_v.at[p], osl(dst_out, ci), swb[p])

        def wb_wait(ci, p):
            pltpu.make_async_copy(a_v.at[p], osl(src_out, ci), swa[p]).wait()
            pltpu.make_async_copy(b_v.at[p], osl(dst_out, ci), swb[p]).wait()

        n_pairs = (N_CHUNKS - 1) // 2
        g_start(0, 0)
        g_start(1, 1)

        def pair_body(i, carry):
            g0 = 2 * i
            g_wait(g0, 0)
            wb_start(g0, 0)
            wb_wait(g0, 0)
            g_start(g0 + 2, 0)

            g_wait(g0 + 1, 1)
            wb_start(g0 + 1, 1)
            wb_wait(g0 + 1, 1)

            @pl.when(i < n_pairs - 1)
            def _():
                g_start(g0 + 3, 1)
            return carry

        lax.fori_loop(0, n_pairs, pair_body, 0)

        last = N_CHUNKS - 1
        g_wait(last, 0)
        wb_start(last, 0)
        wb_wait(last, 0)

    return relay


_relay_cache = []


def _relay(z32, idx0, idx1):
    if not _relay_cache:
        _relay_cache.append(_make_relay())
    return _relay_cache[0](z32, idx0, idx1)


BE = 4000


def _mlp_body(s_ref, d_ref, w1_ref, b1_ref, w2t_ref, b2_ref, o_ref):
    ws = s_ref[...]
    wd = d_ref[...]
    himask = jnp.int32(-65536)
    s_lo = lax.bitcast_convert_type(lax.shift_left(ws, 16), jnp.float32)
    s_hi = lax.bitcast_convert_type(ws & himask, jnp.float32)
    d_lo = lax.bitcast_convert_type(lax.shift_left(wd, 16), jnp.float32)
    d_hi = lax.bitcast_convert_type(wd & himask, jnp.float32)
    xe = (s_lo * d_lo).astype(jnp.bfloat16)
    xo = (s_hi * d_hi).astype(jnp.bfloat16)
    x = jnp.concatenate([xe, xo], axis=1)
    h = jnp.dot(x, w1_ref[...], preferred_element_type=jnp.float32)
    h = jnp.maximum(h + b1_ref[...], 0.0)
    logits = jnp.sum(h * w2t_ref[...], axis=1, keepdims=True) + b2_ref[...]
    o_ref[...] = jax.nn.sigmoid(logits)


def _mlp(s32, d32, W1eo, b1, W2, b2):
    grid = E_SPLIT // BE
    return pl.pallas_call(
        _mlp_body,
        grid=(grid,),
        in_specs=[
            pl.BlockSpec((BE, DW), lambda i: (i, 0)),
            pl.BlockSpec((BE, DW), lambda i: (i, 0)),
            pl.BlockSpec((D, HID), lambda i: (0, 0)),
            pl.BlockSpec((1, HID), lambda i: (0, 0)),
            pl.BlockSpec((1, HID), lambda i: (0, 0)),
            pl.BlockSpec((1, 1), lambda i: (0, 0)),
        ],
        out_specs=pl.BlockSpec((BE, 1), lambda i: (i, 0)),
        out_shape=jax.ShapeDtypeStruct((E_SPLIT, 1), jnp.float32),
        compiler_params=pltpu.CompilerParams(
            dimension_semantics=("arbitrary",),
        ),
    )(s32, d32, W1eo, b1, W2, b2)


def kernel(z, edge, W1, b1, W2, b2):
    zb = z.astype(jnp.bfloat16)
    z32 = lax.bitcast_convert_type(zb.reshape(N_NODES, DW, 2), jnp.int32)
    edge = edge.astype(jnp.int32)
    idx0 = edge[0].reshape(NSPLIT, NW, N_CHUNKS, CHUNK)
    idx1 = edge[1].reshape(NSPLIT, NW, N_CHUNKS, CHUNK)
    w1eo = jnp.concatenate([W1[0::2], W1[1::2]], axis=0).astype(jnp.bfloat16)
    b1r = b1.reshape(1, HID)
    w2r = W2.reshape(1, HID)
    b2r = b2.reshape(1, 1)
    outs = []
    for k in range(NSPLIT):
        s32, d32 = _relay(z32, idx0[k], idx1[k])
        outs.append(_mlp(s32, d32, w1eo, b1r, w2r, b2r))
    if NSPLIT == 1:
        return outs[0]
    return jnp.concatenate(outs, axis=0)


# speedup vs baseline: 1.0956x; 1.0424x over previous
import functools

import jax
import jax.numpy as jnp
from jax import lax
from jax.experimental import pallas as pl
from jax.experimental.pallas import tpu as pltpu
from jax.experimental.pallas import tpu_sc as plsc

N_NODES = 10000
N_EDGES = 160000
D = 256
DW = D // 2
HID = 512

NC = 2
NS = 16
NW = NC * NS
NSPLIT = 5
E_SPLIT = N_EDGES // NSPLIT
E_PER_W = E_SPLIT // NW
CHUNK = 40
N_CHUNKS = E_PER_W // CHUNK
LANES = 16


def _make_relay():
    mesh = plsc.VectorSubcoreMesh(core_axis_name="c", subcore_axis_name="s")

    @functools.partial(
        pl.kernel,
        mesh=mesh,
        out_type=(
            jax.ShapeDtypeStruct((E_SPLIT, DW), jnp.int32),
            jax.ShapeDtypeStruct((E_SPLIT, DW), jnp.int32),
        ),
        compiler_params=pltpu.CompilerParams(
            use_tc_tiling_on_sc=False,
        ),
        scratch_types=[
            pltpu.VMEM((N_CHUNKS, CHUNK), jnp.int32),
            pltpu.VMEM((N_CHUNKS, CHUNK), jnp.int32),
            pltpu.VMEM((2, CHUNK, DW), jnp.int32),
            pltpu.VMEM((2, CHUNK, DW), jnp.int32),
            pltpu.SemaphoreType.DMA,
            pltpu.SemaphoreType.DMA,
            pltpu.SemaphoreType.DMA,
            pltpu.SemaphoreType.DMA,
            pltpu.SemaphoreType.DMA,
            pltpu.SemaphoreType.DMA,
            pltpu.SemaphoreType.DMA,
            pltpu.SemaphoreType.DMA,
        ],
    )
    def relay(z32_hbm, idx0_hbm, idx1_hbm, src_out, dst_out,
              idx0_v, idx1_v, a_v, b_v,
              sga0, sga1, sgb0, sgb1, swa0, swa1, swb0, swb1):
        wid = lax.axis_index("s") * NC + lax.axis_index("c")
        base = wid * E_PER_W
        sga = (sga0, sga1)
        sgb = (sgb0, sgb1)
        swa = (swa0, swa1)
        swb = (swb0, swb1)

        pltpu.sync_copy(idx0_hbm.at[wid], idx0_v)
        pltpu.sync_copy(idx1_hbm.at[wid], idx1_v)

        def g_start(ci, p):
            pltpu.async_copy(z32_hbm.at[idx0_v.at[ci]], a_v.at[p], sga[p])
            pltpu.async_copy(z32_hbm.at[idx1_v.at[ci]], b_v.at[p], sgb[p])

        def g_wait(ci, p):
            pltpu.make_async_copy(
                z32_hbm.at[idx0_v.at[ci]], a_v.at[p], sga[p]).wait()
            pltpu.make_async_copy(
                z32_hbm.at[idx1_v.at[ci]], b_v.at[p], sgb[p]).wait()

        def osl(out, ci):
            return out.at[pl.ds(base + ci * CHUNK, CHUNK)]

        def wb_start(ci, p):
            pltpu.async_copy(a_v.at[p], osl(src_out, ci), swa[p])
            pltpu.async_copy(b_v.at[p], osl(dst_out, ci), swb[p])

        def wb_wait(ci, p):
            pltpu.make_async_copy(a_v.at[p], osl(src_out, ci), swa[p]).wait()
            pltpu.make_async_copy(b_v.at[p], osl(dst_out, ci), swb[p]).wait()

        n_pairs = (N_CHUNKS - 1) // 2
        g_start(0, 0)
        g_start(1, 1)

        def pair_body(i, carry):
            g0 = 2 * i
            g_wait(g0, 0)
            wb_start(g0, 0)
            wb_wait(g0, 0)
            g_start(g0 + 2, 0)

            g_wait(g0 + 1, 1)
            wb_start(g0 + 1, 1)
            wb_wait(g0 + 1, 1)

            @pl.when(i < n_pairs - 1)
            def _():
                g_start(g0 + 3, 1)
            return carry

        lax.fori_loop(0, n_pairs, pair_body, 0)

        last = N_CHUNKS - 1
        g_wait(last, 0)
        wb_start(last, 0)
        wb_wait(last, 0)

    return relay


_relay_cache = []


def _relay(z32, idx0, idx1):
    if not _relay_cache:
        _relay_cache.append(_make_relay())
    return _relay_cache[0](z32, idx0, idx1)


BE = 4000


def _mlp_body(s_ref, d_ref, w1_ref, b1_ref, w2t_ref, b2_ref, o_ref):
    ws = s_ref[...]
    wd = d_ref[...]
    himask = jnp.int32(-65536)
    s_lo = lax.bitcast_convert_type(lax.shift_left(ws, 16), jnp.float32)
    s_hi = lax.bitcast_convert_type(ws & himask, jnp.float32)
    d_lo = lax.bitcast_convert_type(lax.shift_left(wd, 16), jnp.float32)
    d_hi = lax.bitcast_convert_type(wd & himask, jnp.float32)
    xe = (s_lo * d_lo).astype(jnp.bfloat16)
    xo = (s_hi * d_hi).astype(jnp.bfloat16)
    x = jnp.concatenate([xe, xo], axis=1)
    h = jnp.dot(x, w1_ref[...], preferred_element_type=jnp.float32)
    h = jnp.maximum(h + b1_ref[...], 0.0)
    logits = jnp.sum(h * w2t_ref[...], axis=1, keepdims=True) + b2_ref[...]
    o_ref[...] = jax.nn.sigmoid(logits)


def _mlp(s32, d32, W1eo, b1, W2, b2):
    grid = E_SPLIT // BE
    return pl.pallas_call(
        _mlp_body,
        grid=(grid,),
        in_specs=[
            pl.BlockSpec((BE, DW), lambda i: (i, 0)),
            pl.BlockSpec((BE, DW), lambda i: (i, 0)),
            pl.BlockSpec((D, HID), lambda i: (0, 0)),
            pl.BlockSpec((1, HID), lambda i: (0, 0)),
            pl.BlockSpec((1, HID), lambda i: (0, 0)),
            pl.BlockSpec((1, 1), lambda i: (0, 0)),
        ],
        out_specs=pl.BlockSpec((BE, 1), lambda i: (i, 0)),
        out_shape=jax.ShapeDtypeStruct((E_SPLIT, 1), jnp.float32),
        compiler_params=pltpu.CompilerParams(
            dimension_semantics=("arbitrary",),
        ),
    )(s32, d32, W1eo, b1, W2, b2)


def kernel(z, edge, W1, b1, W2, b2):
    zb = z.astype(jnp.bfloat16)
    z32 = lax.bitcast_convert_type(zb.reshape(N_NODES, DW, 2), jnp.int32)
    edge = edge.astype(jnp.int32)
    idx0 = edge[0].reshape(NSPLIT, NW, N_CHUNKS, CHUNK)
    idx1 = edge[1].reshape(NSPLIT, NW, N_CHUNKS, CHUNK)
    w1eo = jnp.concatenate([W1[0::2], W1[1::2]], axis=0).astype(jnp.bfloat16)
    b1r = b1.reshape(1, HID)
    w2r = W2.reshape(1, HID)
    b2r = b2.reshape(1, 1)
    outs = []
    for k in range(NSPLIT):
        s32, d32 = _relay(z32, idx0[k], idx1[k])
        outs.append(_mlp(s32, d32, w1eo, b1r, w2r, b2r))
    if NSPLIT == 1:
        return outs[0]
    return jnp.concatenate(outs, axis=0)


# speedup vs baseline: 1.1876x; 1.0840x over previous
import functools

import jax
import jax.numpy as jnp
from jax import lax
from jax.experimental import pallas as pl
from jax.experimental.pallas import tpu as pltpu
from jax.experimental.pallas import tpu_sc as plsc

N_NODES = 10000
N_EDGES = 160000
D = 256
HID = 512

NC = 2
NS = 16
NW = NC * NS
NSPLIT = 5
E_SPLIT = N_EDGES // NSPLIT
E_PER_W = E_SPLIT // NW
CHUNK = 40
N_CHUNKS = E_PER_W // CHUNK
LANES = 16


def _make_gather_mul():
    mesh = plsc.VectorSubcoreMesh(core_axis_name="c", subcore_axis_name="s")

    @functools.partial(
        pl.kernel,
        mesh=mesh,
        out_type=jax.ShapeDtypeStruct((E_SPLIT, D), jnp.float32),
        scratch_types=[
            pltpu.VMEM((N_CHUNKS, CHUNK), jnp.int32),
            pltpu.VMEM((N_CHUNKS, CHUNK), jnp.int32),
            pltpu.VMEM((2, CHUNK, D), jnp.float32),
            pltpu.VMEM((2, CHUNK, D), jnp.float32),
            pltpu.VMEM((2, CHUNK, D), jnp.float32),
            pltpu.SemaphoreType.DMA,
            pltpu.SemaphoreType.DMA,
            pltpu.SemaphoreType.DMA,
            pltpu.SemaphoreType.DMA,
            pltpu.SemaphoreType.DMA,
            pltpu.SemaphoreType.DMA,
        ],
    )
    def gather_mul(z_hbm, idx0_hbm, idx1_hbm, out_hbm,
                   idx0_v, idx1_v, a_v, b_v, o_v,
                   sa0, sa1, sb0, sb1, so0, so1):
        wid = lax.axis_index("s") * NC + lax.axis_index("c")
        base = wid * E_PER_W
        sa = (sa0, sa1)
        sb = (sb0, sb1)
        so = (so0, so1)

        pltpu.sync_copy(idx0_hbm.at[wid], idx0_v)
        pltpu.sync_copy(idx1_hbm.at[wid], idx1_v)

        def start_gather(ci, p):
            pltpu.async_copy(z_hbm.at[idx0_v.at[ci]], a_v.at[p], sa[p])
            pltpu.async_copy(z_hbm.at[idx1_v.at[ci]], b_v.at[p], sb[p])

        def wait_gather(ci, p):
            pltpu.make_async_copy(z_hbm.at[idx0_v.at[ci]], a_v.at[p], sa[p]).wait()
            pltpu.make_async_copy(z_hbm.at[idx1_v.at[ci]], b_v.at[p], sb[p]).wait()

        def out_slice(ci):
            return out_hbm.at[pl.ds(base + ci * CHUNK, CHUNK)]

        def mul_pack(p):
            def row_body(r, c):
                for k in range(D // LANES):
                    sl = pl.ds(LANES * k, LANES)
                    o_v[p, r, sl] = a_v[p, r, sl] * b_v[p, r, sl]
                return c
            lax.fori_loop(0, CHUNK, row_body, 0)

        def wait_wb(ci, p):
            pltpu.make_async_copy(o_v.at[p], out_slice(ci), so[p]).wait()

        start_gather(0, 0)

        def pair_body(i, carry):
            g0 = 2 * i
            wait_gather(g0, 0)
            start_gather(g0 + 1, 1)

            @pl.when(i > 0)
            def _():
                wait_wb(g0 - 2, 0)
            mul_pack(0)
            pltpu.async_copy(o_v.at[0], out_slice(g0), so[0])

            wait_gather(g0 + 1, 1)
            start_gather(g0 + 2, 0)

            @pl.when(i > 0)
            def _():
                wait_wb(g0 - 1, 1)
            mul_pack(1)
            pltpu.async_copy(o_v.at[1], out_slice(g0 + 1), so[1])
            return carry

        lax.fori_loop(0, (N_CHUNKS - 1) // 2, pair_body, 0)

        last = N_CHUNKS - 1
        wait_gather(last, 0)
        wait_wb(last - 2, 0)
        mul_pack(0)
        pltpu.async_copy(o_v.at[0], out_slice(last), so[0])
        wait_wb(last, 0)
        wait_wb(last - 1, 1)

    return gather_mul


_gather_mul_cache = []


def _gather_mul(z, idx0, idx1):
    if not _gather_mul_cache:
        _gather_mul_cache.append(_make_gather_mul())
    return _gather_mul_cache[0](z, idx0, idx1)


BE = 8000


def _mlp_body(x_ref, w1_ref, b1_ref, w2t_ref, b2_ref, o_ref):
    x = x_ref[...].astype(jnp.bfloat16)
    h = jnp.dot(x, w1_ref[...], preferred_element_type=jnp.float32)
    h = jnp.maximum(h + b1_ref[...], 0.0)
    logits = jnp.sum(h * w2t_ref[...], axis=1, keepdims=True) + b2_ref[...]
    o_ref[...] = jax.nn.sigmoid(logits)


def _mlp(x, W1, b1, W2, b2):
    grid = E_SPLIT // BE
    return pl.pallas_call(
        _mlp_body,
        grid=(grid,),
        in_specs=[
            pl.BlockSpec((BE, D), lambda i: (i, 0)),
            pl.BlockSpec((D, HID), lambda i: (0, 0)),
            pl.BlockSpec((1, HID), lambda i: (0, 0)),
            pl.BlockSpec((1, HID), lambda i: (0, 0)),
            pl.BlockSpec((1, 1), lambda i: (0, 0)),
        ],
        out_specs=pl.BlockSpec((BE, 1), lambda i: (i, 0)),
        out_shape=jax.ShapeDtypeStruct((E_SPLIT, 1), jnp.float32),
        compiler_params=pltpu.CompilerParams(
            dimension_semantics=("arbitrary",),
        ),
    )(x, W1, b1, W2, b2)


def kernel(z, edge, W1, b1, W2, b2):
    edge = edge.astype(jnp.int32)
    idx0 = edge[0].reshape(NSPLIT, NW, N_CHUNKS, CHUNK)
    idx1 = edge[1].reshape(NSPLIT, NW, N_CHUNKS, CHUNK)
    w1 = W1.astype(jnp.bfloat16)
    b1r = b1.reshape(1, HID)
    w2r = W2.reshape(1, HID)
    b2r = b2.reshape(1, 1)
    outs = []
    for k in range(NSPLIT):
        x = _gather_mul(z, idx0[k], idx1[k])
        outs.append(_mlp(x, w1, b1r, w2r, b2r))
    return jnp.concatenate(outs, axis=0)


# speedup vs baseline: 1.2014x; 1.0115x over previous
import functools

import jax
import jax.numpy as jnp
from jax import lax
from jax.experimental import pallas as pl
from jax.experimental.pallas import tpu as pltpu
from jax.experimental.pallas import tpu_sc as plsc

N_NODES = 10000
N_EDGES = 160000
D = 256
HID = 512

NC = 2
NS = 16
NW = NC * NS
NSPLIT = 5
E_SPLIT = N_EDGES // NSPLIT
E_PER_W = E_SPLIT // NW
CHUNK = 40
N_CHUNKS = E_PER_W // CHUNK
LANES = 16


def _make_gather_mul():
    mesh = plsc.VectorSubcoreMesh(core_axis_name="c", subcore_axis_name="s")

    @functools.partial(
        pl.kernel,
        mesh=mesh,
        out_type=jax.ShapeDtypeStruct((E_SPLIT, D), jnp.float32),
        scratch_types=[
            pltpu.VMEM((N_CHUNKS, CHUNK), jnp.int32),
            pltpu.VMEM((N_CHUNKS, CHUNK), jnp.int32),
            pltpu.VMEM((2, CHUNK, D), jnp.float32),
            pltpu.VMEM((2, CHUNK, D), jnp.float32),
            pltpu.VMEM((2, CHUNK, D), jnp.float32),
            pltpu.SemaphoreType.DMA,
            pltpu.SemaphoreType.DMA,
            pltpu.SemaphoreType.DMA,
            pltpu.SemaphoreType.DMA,
            pltpu.SemaphoreType.DMA,
            pltpu.SemaphoreType.DMA,
        ],
    )
    def gather_mul(z_hbm, idx0_hbm, idx1_hbm, out_hbm,
                   idx0_v, idx1_v, a_v, b_v, o_v,
                   sa0, sa1, sb0, sb1, so0, so1):
        wid = lax.axis_index("s") * NC + lax.axis_index("c")
        base = wid * E_PER_W
        sa = (sa0, sa1)
        sb = (sb0, sb1)
        so = (so0, so1)

        pltpu.sync_copy(idx0_hbm.at[wid], idx0_v)
        pltpu.sync_copy(idx1_hbm.at[wid], idx1_v)

        def start_gather(ci, p):
            pltpu.async_copy(z_hbm.at[idx0_v.at[ci]], a_v.at[p], sa[p])
            pltpu.async_copy(z_hbm.at[idx1_v.at[ci]], b_v.at[p], sb[p])

        def wait_gather(ci, p):
            pltpu.make_async_copy(z_hbm.at[idx0_v.at[ci]], a_v.at[p], sa[p]).wait()
            pltpu.make_async_copy(z_hbm.at[idx1_v.at[ci]], b_v.at[p], sb[p]).wait()

        def out_slice(ci):
            return out_hbm.at[pl.ds(base + ci * CHUNK, CHUNK)]

        def mul_pack(p):
            def row_body(r, c):
                for k in range(D // LANES):
                    sl = pl.ds(LANES * k, LANES)
                    o_v[p, r, sl] = a_v[p, r, sl] * b_v[p, r, sl]
                return c
            lax.fori_loop(0, CHUNK, row_body, 0)

        def wait_wb(ci, p):
            pltpu.make_async_copy(o_v.at[p], out_slice(ci), so[p]).wait()

        start_gather(0, 0)

        def pair_body(i, carry):
            g0 = 2 * i
            wait_gather(g0, 0)
            start_gather(g0 + 1, 1)

            @pl.when(i > 0)
            def _():
                wait_wb(g0 - 2, 0)
            mul_pack(0)
            pltpu.async_copy(o_v.at[0], out_slice(g0), so[0])

            wait_gather(g0 + 1, 1)
            start_gather(g0 + 2, 0)

            @pl.when(i > 0)
            def _():
                wait_wb(g0 - 1, 1)
            mul_pack(1)
            pltpu.async_copy(o_v.at[1], out_slice(g0 + 1), so[1])
            return carry

        lax.fori_loop(0, (N_CHUNKS - 1) // 2, pair_body, 0)

        last = N_CHUNKS - 1
        wait_gather(last, 0)
        wait_wb(last - 2, 0)
        mul_pack(0)
        pltpu.async_copy(o_v.at[0], out_slice(last), so[0])
        wait_wb(last, 0)
        wait_wb(last - 1, 1)

    return gather_mul


_gather_mul_cache = []


def _gather_mul(z, idx0, idx1):
    if not _gather_mul_cache:
        _gather_mul_cache.append(_make_gather_mul())
    return _gather_mul_cache[0](z, idx0, idx1)


BE = 16000


def _mlp_body(x_ref, w1_ref, b1_ref, w2t_ref, b2_ref, o_ref):
    x = x_ref[...].astype(jnp.bfloat16)
    h = jnp.dot(x, w1_ref[...], preferred_element_type=jnp.float32)
    h = jnp.maximum(h + b1_ref[...], 0.0)
    logits = jnp.sum(h * w2t_ref[...], axis=1, keepdims=True) + b2_ref[...]
    o_ref[...] = jax.nn.sigmoid(logits)


def _mlp(x, W1, b1, W2, b2):
    grid = E_SPLIT // BE
    return pl.pallas_call(
        _mlp_body,
        grid=(grid,),
        in_specs=[
            pl.BlockSpec((BE, D), lambda i: (i, 0)),
            pl.BlockSpec((D, HID), lambda i: (0, 0)),
            pl.BlockSpec((1, HID), lambda i: (0, 0)),
            pl.BlockSpec((1, HID), lambda i: (0, 0)),
            pl.BlockSpec((1, 1), lambda i: (0, 0)),
        ],
        out_specs=pl.BlockSpec((BE, 1), lambda i: (i, 0)),
        out_shape=jax.ShapeDtypeStruct((E_SPLIT, 1), jnp.float32),
        compiler_params=pltpu.CompilerParams(
            dimension_semantics=("arbitrary",),
        ),
    )(x, W1, b1, W2, b2)


def kernel(z, edge, W1, b1, W2, b2):
    edge = edge.astype(jnp.int32)
    idx0 = edge[0].reshape(NSPLIT, NW, N_CHUNKS, CHUNK)
    idx1 = edge[1].reshape(NSPLIT, NW, N_CHUNKS, CHUNK)
    w1 = W1.astype(jnp.bfloat16)
    b1r = b1.reshape(1, HID)
    w2r = W2.reshape(1, HID)
    b2r = b2.reshape(1, 1)
    outs = []
    for k in range(NSPLIT):
        x = _gather_mul(z, idx0[k], idx1[k])
        outs.append(_mlp(x, w1, b1r, w2r, b2r))
    return jnp.concatenate(outs, axis=0)
